# baseline (speedup 1.0000x reference)
import jax
import jax.numpy as jnp
from jax import lax
from jax.experimental import pallas as pl
from jax.experimental.pallas import tpu as pltpu

VOCAB = 100000
HIDDEN = 128
N_TOK = 512

_VT = 10240


def _body(idx_ref, we_ref, w_ref, b_ref, o_ref, h_raw, h_bf, sem_a, sem_b):
    v = pl.program_id(0)
    half = N_TOK // 2

    @pl.when(v == 0)
    def _gather():
        def issue(i, _):
            pltpu.make_async_copy(
                we_ref.at[pl.ds(idx_ref[i], 1), :], h_raw.at[pl.ds(i, 1), :], sem_a
            ).start()
            pltpu.make_async_copy(
                we_ref.at[pl.ds(idx_ref[i + half], 1), :],
                h_raw.at[pl.ds(i + half, 1), :],
                sem_b,
            ).start()
            return 0

        lax.fori_loop(0, half, issue, 0, unroll=16)
        pltpu.make_async_copy(
            we_ref.at[pl.ds(0, half), :], h_raw.at[pl.ds(0, half), :], sem_a
        ).wait()
        pltpu.make_async_copy(
            we_ref.at[pl.ds(0, half), :], h_raw.at[pl.ds(half, half), :], sem_b
        ).wait()
        h_bf[...] = h_raw[...].astype(jnp.bfloat16)

    w = w_ref[...].astype(jnp.bfloat16)
    acc = lax.dot_general(
        h_bf[...], w, (((1,), (1,)), ((), ())), preferred_element_type=jnp.float32
    )
    o_ref[...] = acc + b_ref[...]


def kernel(x, we, W, b):
    bsz, seq = x.shape
    idx = x.reshape(N_TOK).astype(jnp.int32)
    grid = (pl.cdiv(VOCAB, _VT),)
    out = pl.pallas_call(
        _body,
        grid_spec=pltpu.PrefetchScalarGridSpec(
            num_scalar_prefetch=1,
            grid=grid,
            in_specs=[
                pl.BlockSpec(memory_space=pltpu.HBM),
                pl.BlockSpec((_VT, HIDDEN), lambda v, idx: (v, 0)),
                pl.BlockSpec((1, _VT), lambda v, idx: (0, v)),
            ],
            out_specs=pl.BlockSpec((N_TOK, _VT), lambda v, idx: (0, v)),
            scratch_shapes=[
                pltpu.VMEM((N_TOK, HIDDEN), jnp.float32),
                pltpu.VMEM((N_TOK, HIDDEN), jnp.bfloat16),
                pltpu.SemaphoreType.DMA,
                pltpu.SemaphoreType.DMA,
            ],
        ),
        out_shape=jax.ShapeDtypeStruct((N_TOK, VOCAB), jnp.float32),
        compiler_params=pltpu.CompilerParams(
            dimension_semantics=("arbitrary",),
        ),
    )(idx, we, W, b.reshape(1, VOCAB))
    return out.reshape(bsz, seq, VOCAB)
